# Initial kernel scaffold; baseline (speedup 1.0000x reference)
import jax
import jax.numpy as jnp
from jax.experimental import pallas as pl


def kernel(x, edge_index, edge_type, W1, root1, b1, W2, root2, b2, W3, root3, b3):
    raise NotImplementedError("write your pallas kernel here")


# speedup vs baseline: 9.4771x; 9.4771x over previous
import jax
import jax.numpy as jnp
from jax import lax
from jax.experimental import pallas as pl
from jax.experimental.pallas import tpu as pltpu
from jax.experimental.pallas import tpu_sc as plsc

N = 10000
E = 320000
D = 128
H = 128
R = 8

NP_ = 10112
STRIPE = 632
DUMP = 10000
B = 128
E_PAD = E + 2048
NC, NS = 2, 16
BN = 200


def _sc_body(with_counts, xa, xb, srcp, dstp, etp, starts, sums, hcnt,
             accum, cacc, starts_v, src_v, dst_v, et_v,
             rows_v, ones_v, zbuf, zbuf_c, sem):
    cid = lax.axis_index("c")
    sid = lax.axis_index("s")
    is0 = cid == 0

    def zb_body(i, carry):
        for j in range(4):
            zbuf[i, pl.ds(j * 16, 16)] = jnp.zeros((16,), jnp.float32)
        return carry
    lax.fori_loop(0, 79, zb_body, 0)
    if with_counts:
        def zc_body(i, carry):
            zbuf_c[i] = jnp.zeros((16,), jnp.float32)
            return carry
        lax.fori_loop(0, 79, zc_body, 0)

        one_row = jnp.where(lax.iota(jnp.int32, 16) == 0,
                            jnp.float32(1.0), jnp.float32(0.0))

        def on_body(i, carry):
            ones_v[i] = one_row
            return carry
        lax.fori_loop(0, B, on_body, 0)

    pltpu.sync_copy(starts, starts_v)
    svec = starts_v[...]

    row0 = sid * STRIPE

    for r in range(R):
        s = svec[r]
        e = svec[r + 1]
        a = jnp.bitwise_and(s, jnp.int32(-8))
        per = ((e - a + NS * B - 1) // (NS * B)) * B
        nb = per // B
        t0 = a + sid * per

        for k in range(8):
            pltpu.sync_copy(zbuf, accum.at[pl.ds(row0 + k * 79, 79), :])
        if with_counts:
            @pl.when(is0)
            def _():
                for k in range(8):
                    pltpu.sync_copy(zbuf_c, cacc.at[pl.ds(row0 + k * 79, 79), :])
        plsc.subcore_barrier()

        def batch(j, carry):
            bs = pl.multiple_of(t0 + j * B, 8)
            pltpu.sync_copy(srcp.at[pl.ds(bs, B)], src_v)
            pltpu.sync_copy(dstp.at[pl.ds(bs, B)], dst_v)
            pltpu.sync_copy(etp.at[pl.ds(bs, B)], et_v)

            @pl.when(is0)
            def _():
                pltpu.async_copy(xa.at[src_v], rows_v, sem).wait()

            @pl.when(jnp.logical_not(is0))
            def _():
                pltpu.async_copy(xb.at[src_v], rows_v, sem).wait()
            for j16 in range(B // 16):
                sl = pl.ds(j16 * 16, 16)
                t = et_v[sl]
                d = dst_v[sl]
                dst_v[sl] = jnp.where(t == r, d, jnp.int32(DUMP))
            pltpu.sync_copy(rows_v, accum.at[dst_v], add=True)
            if with_counts:
                @pl.when(is0)
                def _():
                    pltpu.sync_copy(ones_v, cacc.at[dst_v], add=True)
            return carry
        lax.fori_loop(0, nb, batch, 0)
        plsc.subcore_barrier()

        pltpu.sync_copy(accum.at[pl.ds(row0, STRIPE), :],
                        sums.at[cid, r, pl.ds(row0, STRIPE), :])
        if with_counts:
            @pl.when(is0)
            def _():
                pltpu.sync_copy(cacc.at[pl.ds(row0, STRIPE), :],
                                hcnt.at[r, pl.ds(row0, STRIPE), :])
        plsc.subcore_barrier()


def _make_sc_kernel(with_counts):
    mesh = plsc.VectorSubcoreMesh(core_axis_name="c", subcore_axis_name="s",
                                  num_cores=NC, num_subcores=NS)
    out_type = [jax.ShapeDtypeStruct((NC, R, NP_, 64), jnp.float32)]
    if with_counts:
        out_type.append(jax.ShapeDtypeStruct((R, NP_, 16), jnp.float32))

    if with_counts:
        scratch = [
            pltpu.VMEM_SHARED((NP_, 64), jnp.float32),
            pltpu.VMEM_SHARED((NP_, 16), jnp.float32),
            pltpu.VMEM((16,), jnp.int32),
            pltpu.VMEM((B,), jnp.int32),
            pltpu.VMEM((B,), jnp.int32),
            pltpu.VMEM((B,), jnp.int32),
            pltpu.VMEM((B, 64), jnp.float32),
            pltpu.VMEM((B, 16), jnp.float32),
            pltpu.VMEM((79, 64), jnp.float32),
            pltpu.VMEM((79, 16), jnp.float32),
            pltpu.SemaphoreType.DMA,
        ]

        def body(xa, xb, srcp, dstp, etp, starts, sums, hcnt,
                 accum, cacc, starts_v, src_v, dst_v, et_v, rows_v,
                 ones_v, zbuf, zbuf_c, sem):
            _sc_body(True, xa, xb, srcp, dstp, etp, starts, sums, hcnt,
                     accum, cacc, starts_v, src_v, dst_v, et_v, rows_v,
                     ones_v, zbuf, zbuf_c, sem)
    else:
        scratch = [
            pltpu.VMEM_SHARED((NP_, 64), jnp.float32),
            pltpu.VMEM((16,), jnp.int32),
            pltpu.VMEM((B,), jnp.int32),
            pltpu.VMEM((B,), jnp.int32),
            pltpu.VMEM((B,), jnp.int32),
            pltpu.VMEM((B, 64), jnp.float32),
            pltpu.VMEM((79, 64), jnp.float32),
            pltpu.SemaphoreType.DMA,
        ]

        def body(xa, xb, srcp, dstp, etp, starts, sums,
                 accum, starts_v, src_v, dst_v, et_v, rows_v, zbuf, sem):
            _sc_body(False, xa, xb, srcp, dstp, etp, starts, sums, None,
                     accum, None, starts_v, src_v, dst_v, et_v, rows_v,
                     None, zbuf, None, sem)

    return pl.kernel(body, out_type=tuple(out_type), mesh=mesh,
                     scratch_types=scratch,
                     compiler_params=pltpu.CompilerParams(
                         use_tc_tiling_on_sc=False))


def _tc_layer_body(relu, sums_ref, cnt_ref, ua_ref, ub_ref,
                   W_ref, root_ref, b_ref, oa_ref, ob_ref):
    u = jnp.concatenate([ua_ref[...], ub_ref[...]], axis=1)
    acc = jnp.dot(u, root_ref[...],
                  preferred_element_type=jnp.float32) + b_ref[...]
    c = cnt_ref[...]
    for r in range(R):
        inv = 1.0 / jnp.maximum(c[:, r:r + 1], 1.0)
        m = jnp.concatenate([sums_ref[0, r], sums_ref[1, r]], axis=1)
        acc = acc + jnp.dot(m * inv, W_ref[r],
                            preferred_element_type=jnp.float32)
    if relu:
        acc = jnp.maximum(acc, 0.0)
    oa_ref[...] = acc[:, :64]
    ob_ref[...] = acc[:, 64:]


def _tc_layer3_body(sums_ref, cnt_ref, ua_ref, ub_ref,
                    W_ref, root_ref, b_ref, out_ref):
    u = jnp.concatenate([ua_ref[...], ub_ref[...]], axis=1)
    acc = jnp.dot(u, root_ref[...],
                  preferred_element_type=jnp.float32) + b_ref[...]
    c = cnt_ref[...]
    for r in range(R):
        inv = 1.0 / jnp.maximum(c[:, r:r + 1], 1.0)
        m = jnp.concatenate([sums_ref[0, r], sums_ref[1, r]], axis=1)
        acc = acc + jnp.dot(m * inv, W_ref[r],
                            preferred_element_type=jnp.float32)

    @pl.when(pl.program_id(0) == 0)
    def _():
        out_ref[...] = jnp.zeros_like(out_ref)
    out_ref[...] += jnp.sum(acc, axis=0, keepdims=True) * (1.0 / N)


_IN_SPECS = [
    pl.BlockSpec((NC, R, BN, 64), lambda n: (0, 0, n, 0)),
    pl.BlockSpec((BN, R), lambda n: (n, 0)),
    pl.BlockSpec((BN, 64), lambda n: (n, 0)),
    pl.BlockSpec((BN, 64), lambda n: (n, 0)),
    pl.BlockSpec((R, D, H), lambda n: (0, 0, 0)),
    pl.BlockSpec((D, H), lambda n: (0, 0)),
    pl.BlockSpec((1, H), lambda n: (0, 0)),
]


def _tc_layer(sums, cntT, ua, ub, W, root, b, relu):
    import functools
    return pl.pallas_call(
        functools.partial(_tc_layer_body, relu),
        grid=(N // BN,),
        in_specs=_IN_SPECS,
        out_specs=[pl.BlockSpec((BN, 64), lambda n: (n, 0)),
                   pl.BlockSpec((BN, 64), lambda n: (n, 0))],
        out_shape=[jax.ShapeDtypeStruct((N, 64), jnp.float32),
                   jax.ShapeDtypeStruct((N, 64), jnp.float32)],
        compiler_params=pltpu.CompilerParams(
            dimension_semantics=("arbitrary",)),
    )(sums, cntT, ua, ub, W, root, b)


def _tc_layer3(sums, cntT, ua, ub, W, root, b):
    return pl.pallas_call(
        _tc_layer3_body,
        grid=(N // BN,),
        in_specs=_IN_SPECS,
        out_specs=pl.BlockSpec((1, H), lambda n: (0, 0)),
        out_shape=jax.ShapeDtypeStruct((1, H), jnp.float32),
        compiler_params=pltpu.CompilerParams(
            dimension_semantics=("arbitrary",)),
    )(sums, cntT, ua, ub, W, root, b)


def kernel(x, edge_index, edge_type, W1, root1, b1, W2, root2, b2,
           W3, root3, b3):
    src = edge_index[0].astype(jnp.int32)
    dst = edge_index[1].astype(jnp.int32)
    et = edge_type.astype(jnp.int32)

    starts = jnp.searchsorted(
        et, jnp.arange(R + 1, dtype=jnp.int32)).astype(jnp.int32)
    starts = jnp.concatenate(
        [starts, jnp.full((16 - R - 1,), E, jnp.int32)])
    pad = E_PAD - E
    srcp = jnp.concatenate([src, jnp.zeros((pad,), jnp.int32)])
    dstp = jnp.concatenate([dst, jnp.full((pad,), DUMP, jnp.int32)])
    etp = jnp.concatenate([et, jnp.full((pad,), 99, jnp.int32)])

    xa = x[:, :64]
    xb = x[:, 64:]
    b1r = b1.reshape(1, H)
    b2r = b2.reshape(1, H)
    b3r = b3.reshape(1, H)

    sc_first = _make_sc_kernel(True)
    sc_rest = _make_sc_kernel(False)

    sums1, hcnt = sc_first(xa, xb, srcp, dstp, etp, starts)
    cntT = hcnt[:, :, 0].T

    ua1, ub1 = _tc_layer(sums1, cntT, xa, xb, W1, root1, b1r, True)
    (sums2,) = sc_rest(ua1, ub1, srcp, dstp, etp, starts)
    ua2, ub2 = _tc_layer(sums2, cntT, ua1, ub1, W2, root2, b2r, True)
    (sums3,) = sc_rest(ua2, ub2, srcp, dstp, etp, starts)
    return _tc_layer3(sums3, cntT, ua2, ub2, W3, root3, b3r)


# speedup vs baseline: 13.3052x; 1.4039x over previous
import functools

import jax
import jax.numpy as jnp
from jax import lax
from jax.experimental import pallas as pl
from jax.experimental.pallas import tpu as pltpu
from jax.experimental.pallas import tpu_sc as plsc

N = 10000
E = 320000
D = 128
H = 128
R = 8

NP_ = 10112
STRIPE = 632
DUMP = 10000
B = 128
KC = 4
KR = 8
ZROWS = 158
E_PAD = E + 4096
EROWS = E_PAD // B
NC, NS = 2, 16
BN = 200


def _sc_body(with_counts, K, xa, xb, src2, dst2, et2, starts, sums, hcnt,
             accum, cacc, starts_v, src_v, dst_v, et_v,
             rows_v, ones_v, zbuf, zbuf_c, sem, sem2, sem3):
    SB = K * B
    cid = lax.axis_index("c")
    sid = lax.axis_index("s")
    is0 = cid == 0

    def zb_body(i, carry):
        for j in range(4):
            zbuf[i, pl.ds(j * 16, 16)] = jnp.zeros((16,), jnp.float32)
        return carry
    lax.fori_loop(0, ZROWS, zb_body, 0)
    if with_counts:
        def zc_body(i, carry):
            zbuf_c[i] = jnp.zeros((16,), jnp.float32)
            return carry
        lax.fori_loop(0, ZROWS, zc_body, 0)

        one_row = jnp.where(lax.iota(jnp.int32, 16) == 0,
                            jnp.float32(1.0), jnp.float32(0.0))

        def on_body(i, carry):
            ones_v[i] = one_row
            return carry
        lax.fori_loop(0, B, on_body, 0)

    pltpu.sync_copy(starts, starts_v)
    svec = starts_v[...]

    row0 = sid * STRIPE
    lanes = lax.iota(jnp.int32, 16)

    for r in range(R):
        s = svec[r]
        e = svec[r + 1]
        a = jnp.bitwise_and(s, jnp.int32(-128))
        per = ((e - a + NS * B - 1) // (NS * B)) * B
        nbs = (per // B + K - 1) // K
        t0 = a + sid * per
        t_end = t0 + per
        t0r = t0 // B

        for z in range(STRIPE // ZROWS):
            pltpu.sync_copy(zbuf, accum.at[pl.ds(row0 + z * ZROWS, ZROWS), :])
        if with_counts:
            @pl.when(is0)
            def _():
                for z in range(STRIPE // ZROWS):
                    pltpu.sync_copy(
                        zbuf_c, cacc.at[pl.ds(row0 + z * ZROWS, ZROWS), :])
        plsc.subcore_barrier()

        def sbatch(j, carry):
            rowb = t0r + j * K
            pltpu.sync_copy(src2.at[pl.ds(rowb, K), :], src_v)
            pltpu.sync_copy(dst2.at[pl.ds(rowb, K), :], dst_v)
            pltpu.sync_copy(et2.at[pl.ds(rowb, K), :], et_v)
            bs0 = t0 + j * SB
            for k in range(K):
                for j16 in range(B // 16):
                    sl = pl.ds(j16 * 16, 16)
                    pos = (bs0 + k * B + j16 * 16) + lanes
                    t = et_v[k, sl]
                    d = dst_v[k, sl]
                    ok = jnp.logical_and(t == r, pos < t_end)
                    dst_v[k, sl] = jnp.where(ok, d, jnp.int32(DUMP))

            def gather_all(tab):
                def _fire():
                    descs = [
                        pltpu.async_copy(tab.at[src_v.at[k]],
                                         rows_v.at[pl.ds(k * B, B), :], sem)
                        for k in range(K)]
                    for dsc in descs:
                        dsc.wait()
                return _fire
            pl.when(is0)(gather_all(xa))
            pl.when(jnp.logical_not(is0))(gather_all(xb))

            descs = [
                pltpu.async_copy(rows_v.at[pl.ds(k * B, B), :],
                                 accum.at[dst_v.at[k]], sem2, add=True)
                for k in range(K)]
            if with_counts:
                @pl.when(is0)
                def _():
                    cds = [
                        pltpu.async_copy(ones_v, cacc.at[dst_v.at[k]],
                                         sem3, add=True)
                        for k in range(K)]
                    for dsc in cds:
                        dsc.wait()
            for dsc in descs:
                dsc.wait()
            return carry
        lax.fori_loop(0, nbs, sbatch, 0)
        plsc.subcore_barrier()

        pltpu.sync_copy(accum.at[pl.ds(row0, STRIPE), :],
                        sums.at[cid, r, pl.ds(row0, STRIPE), :])
        if with_counts:
            @pl.when(is0)
            def _():
                pltpu.sync_copy(cacc.at[pl.ds(row0, STRIPE), :],
                                hcnt.at[r, pl.ds(row0, STRIPE), :])
        plsc.subcore_barrier()


def _make_sc_kernel(with_counts):
    mesh = plsc.VectorSubcoreMesh(core_axis_name="c", subcore_axis_name="s",
                                  num_cores=NC, num_subcores=NS)
    out_type = [jax.ShapeDtypeStruct((NC, R, NP_, 64), jnp.float32)]
    if with_counts:
        out_type.append(jax.ShapeDtypeStruct((R, NP_, 16), jnp.float32))

    if with_counts:
        K = KC
        scratch = [
            pltpu.VMEM_SHARED((NP_, 64), jnp.float32),
            pltpu.VMEM_SHARED((NP_, 16), jnp.float32),
            pltpu.VMEM((16,), jnp.int32),
            pltpu.VMEM((K, B), jnp.int32),
            pltpu.VMEM((K, B), jnp.int32),
            pltpu.VMEM((K, B), jnp.int32),
            pltpu.VMEM((K * B, 64), jnp.float32),
            pltpu.VMEM((B, 16), jnp.float32),
            pltpu.VMEM((ZROWS, 64), jnp.float32),
            pltpu.VMEM((ZROWS, 16), jnp.float32),
            pltpu.SemaphoreType.DMA,
            pltpu.SemaphoreType.DMA,
            pltpu.SemaphoreType.DMA,
        ]

        def body(xa, xb, src2, dst2, et2, starts, sums, hcnt,
                 accum, cacc, starts_v, src_v, dst_v, et_v, rows_v,
                 ones_v, zbuf, zbuf_c, sem, sem2, sem3):
            _sc_body(True, KC, xa, xb, src2, dst2, et2, starts, sums, hcnt,
                     accum, cacc, starts_v, src_v, dst_v, et_v, rows_v,
                     ones_v, zbuf, zbuf_c, sem, sem2, sem3)
    else:
        K = KR
        scratch = [
            pltpu.VMEM_SHARED((NP_, 64), jnp.float32),
            pltpu.VMEM((16,), jnp.int32),
            pltpu.VMEM((K, B), jnp.int32),
            pltpu.VMEM((K, B), jnp.int32),
            pltpu.VMEM((K, B), jnp.int32),
            pltpu.VMEM((K * B, 64), jnp.float32),
            pltpu.VMEM((ZROWS, 64), jnp.float32),
            pltpu.SemaphoreType.DMA,
            pltpu.SemaphoreType.DMA,
        ]

        def body(xa, xb, src2, dst2, et2, starts, sums,
                 accum, starts_v, src_v, dst_v, et_v, rows_v, zbuf,
                 sem, sem2):
            _sc_body(False, KR, xa, xb, src2, dst2, et2, starts, sums, None,
                     accum, None, starts_v, src_v, dst_v, et_v, rows_v,
                     None, zbuf, None, sem, sem2, None)

    return pl.kernel(body, out_type=tuple(out_type), mesh=mesh,
                     scratch_types=scratch,
                     compiler_params=pltpu.CompilerParams(
                         use_tc_tiling_on_sc=False))


def _tc_layer_body(relu, sums_ref, cnt_ref, ua_ref, ub_ref,
                   W_ref, root_ref, b_ref, oa_ref, ob_ref):
    u = jnp.concatenate([ua_ref[...], ub_ref[...]], axis=1)
    acc = jnp.dot(u, root_ref[...],
                  preferred_element_type=jnp.float32) + b_ref[...]
    c = cnt_ref[...]
    for r in range(R):
        inv = 1.0 / jnp.maximum(c[:, r:r + 1], 1.0)
        m = jnp.concatenate([sums_ref[0, r], sums_ref[1, r]], axis=1)
        acc = acc + jnp.dot(m * inv, W_ref[r],
                            preferred_element_type=jnp.float32)
    if relu:
        acc = jnp.maximum(acc, 0.0)
    oa_ref[...] = acc[:, :64]
    ob_ref[...] = acc[:, 64:]


def _tc_layer3_body(sums_ref, cnt_ref, ua_ref, ub_ref,
                    W_ref, root_ref, b_ref, out_ref):
    u = jnp.concatenate([ua_ref[...], ub_ref[...]], axis=1)
    acc = jnp.dot(u, root_ref[...],
                  preferred_element_type=jnp.float32) + b_ref[...]
    c = cnt_ref[...]
    for r in range(R):
        inv = 1.0 / jnp.maximum(c[:, r:r + 1], 1.0)
        m = jnp.concatenate([sums_ref[0, r], sums_ref[1, r]], axis=1)
        acc = acc + jnp.dot(m * inv, W_ref[r],
                            preferred_element_type=jnp.float32)

    @pl.when(pl.program_id(0) == 0)
    def _():
        out_ref[...] = jnp.zeros_like(out_ref)
    out_ref[...] += jnp.sum(acc, axis=0, keepdims=True) * (1.0 / N)


_IN_SPECS = [
    pl.BlockSpec((NC, R, BN, 64), lambda n: (0, 0, n, 0)),
    pl.BlockSpec((BN, R), lambda n: (n, 0)),
    pl.BlockSpec((BN, 64), lambda n: (n, 0)),
    pl.BlockSpec((BN, 64), lambda n: (n, 0)),
    pl.BlockSpec((R, D, H), lambda n: (0, 0, 0)),
    pl.BlockSpec((D, H), lambda n: (0, 0)),
    pl.BlockSpec((1, H), lambda n: (0, 0)),
]


def _tc_layer(sums, cntT, ua, ub, W, root, b, relu):
    return pl.pallas_call(
        functools.partial(_tc_layer_body, relu),
        grid=(N // BN,),
        in_specs=_IN_SPECS,
        out_specs=[pl.BlockSpec((BN, 64), lambda n: (n, 0)),
                   pl.BlockSpec((BN, 64), lambda n: (n, 0))],
        out_shape=[jax.ShapeDtypeStruct((N, 64), jnp.float32),
                   jax.ShapeDtypeStruct((N, 64), jnp.float32)],
        compiler_params=pltpu.CompilerParams(
            dimension_semantics=("arbitrary",)),
    )(sums, cntT, ua, ub, W, root, b)


def _tc_layer3(sums, cntT, ua, ub, W, root, b):
    return pl.pallas_call(
        _tc_layer3_body,
        grid=(N // BN,),
        in_specs=_IN_SPECS,
        out_specs=pl.BlockSpec((1, H), lambda n: (0, 0)),
        out_shape=jax.ShapeDtypeStruct((1, H), jnp.float32),
        compiler_params=pltpu.CompilerParams(
            dimension_semantics=("arbitrary",)),
    )(sums, cntT, ua, ub, W, root, b)


def kernel(x, edge_index, edge_type, W1, root1, b1, W2, root2, b2,
           W3, root3, b3):
    src = edge_index[0].astype(jnp.int32)
    dst = edge_index[1].astype(jnp.int32)
    et = edge_type.astype(jnp.int32)

    starts = jnp.searchsorted(
        et, jnp.arange(R + 1, dtype=jnp.int32)).astype(jnp.int32)
    starts = jnp.concatenate(
        [starts, jnp.full((16 - R - 1,), E, jnp.int32)])
    pad = E_PAD - E
    src2 = jnp.concatenate([src, jnp.zeros((pad,), jnp.int32)]).reshape(
        EROWS, B)
    dst2 = jnp.concatenate([dst, jnp.full((pad,), DUMP, jnp.int32)]).reshape(
        EROWS, B)
    et2 = jnp.concatenate([et, jnp.full((pad,), 99, jnp.int32)]).reshape(
        EROWS, B)

    xa = x[:, :64]
    xb = x[:, 64:]
    b1r = b1.reshape(1, H)
    b2r = b2.reshape(1, H)
    b3r = b3.reshape(1, H)

    sc_first = _make_sc_kernel(True)
    sc_rest = _make_sc_kernel(False)

    sums1, hcnt = sc_first(xa, xb, src2, dst2, et2, starts)
    cntT = hcnt[:, :, 0].T

    ua1, ub1 = _tc_layer(sums1, cntT, xa, xb, W1, root1, b1r, True)
    (sums2,) = sc_rest(ua1, ub1, src2, dst2, et2, starts)
    ua2, ub2 = _tc_layer(sums2, cntT, ua1, ub1, W2, root2, b2r, True)
    (sums3,) = sc_rest(ua2, ub2, src2, dst2, et2, starts)
    return _tc_layer3(sums3, cntT, ua2, ub2, W3, root3, b3r)
